# baseline (speedup 1.0000x reference)
import jax
import jax.numpy as jnp
from jax.experimental import pallas as pl
from jax.experimental.pallas import tpu as pltpu

_B, _N, _F, _E, _C = 128, 256, 16, 4, 128
_G = 16
_CH = 64


def _gat_block(adj_ref, nf_ref, w_ref, attsd_ref, wedge_ref, attedge_ref,
               bias_ref, wout_ref, bout_ref, out_ref, gf_ref):
    w = w_ref[...]
    attsd = attsd_ref[...]
    bias = bias_ref[...]
    wout = wout_ref[...]
    bout = bout_ref[...]

    esb = jnp.sum(wedge_ref[...] * attedge_ref[...], axis=1, keepdims=True)
    s1 = esb[1:2, 0:1]
    s2 = esb[2:3, 0:1]
    s3 = esb[3:4, 0:1]

    nf = nf_ref[...].reshape(_G * _N, _F)
    h = jnp.dot(nf, w, preferred_element_type=jnp.float32)
    asd = jnp.dot(h, attsd, preferred_element_type=jnp.float32)

    rows = jax.lax.broadcasted_iota(jnp.int32, (_N, _N), 0)
    cols = jax.lax.broadcasted_iota(jnp.int32, (_N, _N), 1)
    eye = jnp.where(rows == cols, 1.0, 0.0).astype(jnp.float32)
    ones_row = jnp.full((1, _N), 1.0, dtype=jnp.float32)

    ad_cols = jnp.concatenate(
        [asd[g * _N:(g + 1) * _N, 1:2] for g in range(_G)], axis=1)
    as_cols = jnp.concatenate(
        [asd[g * _N:(g + 1) * _N, 0:1] for g in range(_G)], axis=1)
    adas = jnp.concatenate([ad_cols, as_cols], axis=1)
    adas_rows = jax.lax.dot_general(adas, eye, (((0,), (0,)), ((), ())),
                                    preferred_element_type=jnp.float32)
    ad_rows = adas_rows[:_G, :]
    as_max = jnp.max(adas_rows[_G:, :], axis=1, keepdims=True)
    s_max = jnp.maximum(jnp.maximum(s1, s2), s3)

    h16 = h.astype(jnp.bfloat16)

    rs = []
    qs = []
    for g in range(_G):
        adj = adj_ref[g]
        a_s = asd[g * _N:(g + 1) * _N, 0:1]
        a_d = ad_rows[g:g + 1, :]
        mt = jnp.maximum(as_max[g:g + 1, 0:1] + a_d + s_max, 0.0)
        row1 = a_d + s1
        row2 = a_d + s2
        row3 = a_d + s3
        parts = []
        u_chunks = []
        for c in range(_N // _CH):
            adjc = adj[c * _CH:(c + 1) * _CH, :]
            a_sc = a_s[c * _CH:(c + 1) * _CH, :]
            hi = adjc >= 2
            sm = jnp.where(hi,
                           jnp.where(adjc == 2, row2, row3),
                           jnp.where(adjc == 1, row1, -1e30))
            e = a_sc + sm
            e = jnp.maximum(e, 0.2 * e)
            uf = jnp.exp2(e - mt)
            parts.append(jnp.sum(uf.reshape(_CH // 8, 8, _N), axis=0))
            u_chunks.append(uf.astype(jnp.bfloat16))
        u16 = jnp.concatenate(u_chunks, axis=0)
        dacc = (parts[0] + parts[1]) + (parts[2] + parts[3])
        denom = jnp.sum(dacc, axis=0, keepdims=True)
        r = jnp.where(denom <= 1e-35, 0.0, 1.0 / denom)
        rs.append(r)
        hg = h16[g * _N:(g + 1) * _N, :]
        qs.append(jax.lax.dot_general(u16, hg, (((0,), (0,)), ((), ())),
                                      preferred_element_type=jnp.float32))

    q_all = jnp.concatenate(qs, axis=0)
    r_cat = jnp.concatenate(rs, axis=0)
    gidx = jax.lax.broadcasted_iota(jnp.int32, (_G, _N), 0)
    r_blk = jnp.concatenate(
        [jnp.where(gidx == g, r_cat, 0.0) for g in range(_G)], axis=1)
    gf = jnp.dot(r_blk, q_all, preferred_element_type=jnp.float32) * (1.0 / _N) + bias
    out = jnp.dot(gf, wout, preferred_element_type=jnp.float32) + bout
    gf_ref[...] = gf
    out_ref[...] = out


def kernel(adjacency_tensor, node_features, W, att_src, att_dst, W_edge,
           att_edge, bias, W_out, b_out):
    log2e = 1.4426950408889634
    attsd = jnp.stack([att_src, att_dst], axis=1) * log2e
    attedge = att_edge.reshape(1, _C) * log2e
    bias2 = bias.reshape(1, _C)
    bout2 = b_out.reshape(1, 1)

    grid = (_B // _G,)
    out, gf = pl.pallas_call(
        _gat_block,
        grid=grid,
        in_specs=[
            pl.BlockSpec((_G, _N, _N), lambda i: (i, 0, 0)),
            pl.BlockSpec((_G, _N, _F), lambda i: (i, 0, 0)),
            pl.BlockSpec((_F, _C), lambda i: (0, 0)),
            pl.BlockSpec((_C, 2), lambda i: (0, 0)),
            pl.BlockSpec((_E, _C), lambda i: (0, 0)),
            pl.BlockSpec((1, _C), lambda i: (0, 0)),
            pl.BlockSpec((1, _C), lambda i: (0, 0)),
            pl.BlockSpec((_C, 1), lambda i: (0, 0)),
            pl.BlockSpec((1, 1), lambda i: (0, 0)),
        ],
        out_specs=[
            pl.BlockSpec((_G, 1), lambda i: (i, 0)),
            pl.BlockSpec((_G, _C), lambda i: (i, 0)),
        ],
        out_shape=[
            jax.ShapeDtypeStruct((_B, 1), jnp.float32),
            jax.ShapeDtypeStruct((_B, _C), jnp.float32),
        ],
    )(adjacency_tensor, node_features, W, attsd, W_edge, attedge, bias2,
      W_out, bout2)
    return (out, gf)


# speedup vs baseline: 1.0215x; 1.0215x over previous
import jax
import jax.numpy as jnp
from jax.experimental import pallas as pl
from jax.experimental.pallas import tpu as pltpu

_B, _N, _F, _E, _C = 128, 256, 16, 4, 128
_G = 16


def _gat_block(adj_ref, nf_ref, w_ref, attsd_ref, wedge_ref, attedge_ref,
               bias_ref, wout_ref, bout_ref, out_ref, gf_ref):
    w = w_ref[...]
    attsd = attsd_ref[...]
    bias = bias_ref[...]
    wout = wout_ref[...]
    bout = bout_ref[...]

    esb = jnp.sum(wedge_ref[...] * attedge_ref[...], axis=1, keepdims=True)
    s1 = esb[1:2, 0:1]
    s2 = esb[2:3, 0:1]
    s3 = esb[3:4, 0:1]

    nf = nf_ref[...].reshape(_G * _N, _F)
    h = jnp.dot(nf, w, preferred_element_type=jnp.float32)
    asd = jnp.dot(h, attsd, preferred_element_type=jnp.float32)

    rows = jax.lax.broadcasted_iota(jnp.int32, (_N, _N), 0)
    cols = jax.lax.broadcasted_iota(jnp.int32, (_N, _N), 1)
    eye = jnp.where(rows == cols, 1.0, 0.0).astype(jnp.float32)
    ones_row = jnp.full((1, _N), 1.0, dtype=jnp.float32)

    ad_cols = jnp.concatenate(
        [asd[g * _N:(g + 1) * _N, 1:2] for g in range(_G)], axis=1)
    ad_rows = jax.lax.dot_general(ad_cols, eye, (((0,), (0,)), ((), ())),
                                  preferred_element_type=jnp.float32)

    h16 = h.astype(jnp.bfloat16)

    rs = []
    qs = []
    for g in range(_G):
        adj = adj_ref[g]
        a_s = asd[g * _N:(g + 1) * _N, 0:1]
        a_d = ad_rows[g:g + 1, :]
        af = jnp.where(adj == 2, s2, jnp.where(adj == 3, s3, s1))
        e = a_s + a_d + af
        e = jnp.maximum(e, 0.2 * e)
        s = jnp.where(adj > 0, e, -1e30)
        m = jnp.max(s, axis=0, keepdims=True)
        u = jnp.exp(s - m)
        denom = jnp.sum(u, axis=0, keepdims=True)
        r = jnp.where(m < -1e29, 0.0, 1.0 / jnp.maximum(denom, 1e-12))
        rs.append(r)
        hg = h16[g * _N:(g + 1) * _N, :]
        qs.append(jax.lax.dot_general(u.astype(jnp.bfloat16), hg,
                                      (((0,), (0,)), ((), ())),
                                      preferred_element_type=jnp.float32))

    q_all = jnp.concatenate(qs, axis=0)
    r_cat = jnp.concatenate(rs, axis=0)
    gidx = jax.lax.broadcasted_iota(jnp.int32, (_G, _N), 0)
    r_blk = jnp.concatenate(
        [jnp.where(gidx == g, r_cat, 0.0) for g in range(_G)], axis=1)
    gf = jnp.dot(r_blk, q_all, preferred_element_type=jnp.float32) * (1.0 / _N) + bias
    out = jnp.dot(gf, wout, preferred_element_type=jnp.float32) + bout
    gf_ref[...] = gf
    out_ref[...] = out


def kernel(adjacency_tensor, node_features, W, att_src, att_dst, W_edge,
           att_edge, bias, W_out, b_out):
    attsd = jnp.stack([att_src, att_dst], axis=1)
    attedge = att_edge.reshape(1, _C)
    bias2 = bias.reshape(1, _C)
    bout2 = b_out.reshape(1, 1)

    grid = (_B // _G,)
    out, gf = pl.pallas_call(
        _gat_block,
        grid=grid,
        compiler_params=pltpu.CompilerParams(
            dimension_semantics=("parallel",)),
        in_specs=[
            pl.BlockSpec((_G, _N, _N), lambda i: (i, 0, 0)),
            pl.BlockSpec((_G, _N, _F), lambda i: (i, 0, 0)),
            pl.BlockSpec((_F, _C), lambda i: (0, 0)),
            pl.BlockSpec((_C, 2), lambda i: (0, 0)),
            pl.BlockSpec((_E, _C), lambda i: (0, 0)),
            pl.BlockSpec((1, _C), lambda i: (0, 0)),
            pl.BlockSpec((1, _C), lambda i: (0, 0)),
            pl.BlockSpec((_C, 1), lambda i: (0, 0)),
            pl.BlockSpec((1, 1), lambda i: (0, 0)),
        ],
        out_specs=[
            pl.BlockSpec((_G, 1), lambda i: (i, 0)),
            pl.BlockSpec((_G, _C), lambda i: (i, 0)),
        ],
        out_shape=[
            jax.ShapeDtypeStruct((_B, 1), jnp.float32),
            jax.ShapeDtypeStruct((_B, _C), jnp.float32),
        ],
    )(adjacency_tensor, node_features, W, attsd, W_edge, attedge, bias2,
      W_out, bout2)
    return (out, gf)


# speedup vs baseline: 1.0485x; 1.0264x over previous
import jax
import jax.numpy as jnp
from jax.experimental import pallas as pl
from jax.experimental.pallas import tpu as pltpu

_B, _N, _F, _E, _C = 128, 256, 16, 4, 128
_G = 32


def _gat_block(adj_ref, nf_ref, w_ref, attsd_ref, wedge_ref, attedge_ref,
               bias_ref, wout_ref, bout_ref, out_ref, gf_ref):
    w = w_ref[...]
    attsd = attsd_ref[...]
    bias = bias_ref[...]
    wout = wout_ref[...]
    bout = bout_ref[...]

    esb = jnp.sum(wedge_ref[...] * attedge_ref[...], axis=1, keepdims=True)
    s1 = esb[1:2, 0:1]
    s2 = esb[2:3, 0:1]
    s3 = esb[3:4, 0:1]

    nf = nf_ref[...].reshape(_G * _N, _F)
    h = jnp.dot(nf, w, preferred_element_type=jnp.float32)
    asd = jnp.dot(h, attsd, preferred_element_type=jnp.float32)

    rows = jax.lax.broadcasted_iota(jnp.int32, (_N, _N), 0)
    cols = jax.lax.broadcasted_iota(jnp.int32, (_N, _N), 1)
    eye = jnp.where(rows == cols, 1.0, 0.0).astype(jnp.float32)
    ones_row = jnp.full((1, _N), 1.0, dtype=jnp.float32)

    ad_cols = jnp.concatenate(
        [asd[g * _N:(g + 1) * _N, 1:2] for g in range(_G)], axis=1)
    ad_rows = jax.lax.dot_general(ad_cols, eye, (((0,), (0,)), ((), ())),
                                  preferred_element_type=jnp.float32)

    h16 = h.astype(jnp.bfloat16)

    rs = []
    qs = []
    for g in range(_G):
        adj = adj_ref[g]
        a_s = asd[g * _N:(g + 1) * _N, 0:1]
        a_d = ad_rows[g:g + 1, :]
        af = jnp.where(adj == 2, s2, jnp.where(adj == 3, s3, s1))
        e = a_s + a_d + af
        e = jnp.maximum(e, 0.2 * e)
        s = jnp.where(adj > 0, e, -1e30)
        m = jnp.max(s, axis=0, keepdims=True)
        u = jnp.exp(s - m)
        denom = jnp.sum(u, axis=0, keepdims=True)
        r = jnp.where(m < -1e29, 0.0, 1.0 / jnp.maximum(denom, 1e-12))
        rs.append(r)
        hg = h16[g * _N:(g + 1) * _N, :]
        qs.append(jax.lax.dot_general(u.astype(jnp.bfloat16), hg,
                                      (((0,), (0,)), ((), ())),
                                      preferred_element_type=jnp.float32))

    q_all = jnp.concatenate(qs, axis=0)
    r_cat = jnp.concatenate(rs, axis=0)
    gidx = jax.lax.broadcasted_iota(jnp.int32, (_G, _N), 0)
    r_blk = jnp.concatenate(
        [jnp.where(gidx == g, r_cat, 0.0) for g in range(_G)], axis=1)
    gf = jnp.dot(r_blk, q_all, preferred_element_type=jnp.float32) * (1.0 / _N) + bias
    out = jnp.dot(gf, wout, preferred_element_type=jnp.float32) + bout
    gf_ref[...] = gf
    out_ref[...] = out


def kernel(adjacency_tensor, node_features, W, att_src, att_dst, W_edge,
           att_edge, bias, W_out, b_out):
    attsd = jnp.stack([att_src, att_dst], axis=1)
    attedge = att_edge.reshape(1, _C)
    bias2 = bias.reshape(1, _C)
    bout2 = b_out.reshape(1, 1)

    grid = (_B // _G,)
    out, gf = pl.pallas_call(
        _gat_block,
        grid=grid,
        compiler_params=pltpu.CompilerParams(
            dimension_semantics=("parallel",)),
        in_specs=[
            pl.BlockSpec((_G, _N, _N), lambda i: (i, 0, 0)),
            pl.BlockSpec((_G, _N, _F), lambda i: (i, 0, 0)),
            pl.BlockSpec((_F, _C), lambda i: (0, 0)),
            pl.BlockSpec((_C, 2), lambda i: (0, 0)),
            pl.BlockSpec((_E, _C), lambda i: (0, 0)),
            pl.BlockSpec((1, _C), lambda i: (0, 0)),
            pl.BlockSpec((1, _C), lambda i: (0, 0)),
            pl.BlockSpec((_C, 1), lambda i: (0, 0)),
            pl.BlockSpec((1, 1), lambda i: (0, 0)),
        ],
        out_specs=[
            pl.BlockSpec((_G, 1), lambda i: (i, 0)),
            pl.BlockSpec((_G, _C), lambda i: (i, 0)),
        ],
        out_shape=[
            jax.ShapeDtypeStruct((_B, 1), jnp.float32),
            jax.ShapeDtypeStruct((_B, _C), jnp.float32),
        ],
    )(adjacency_tensor, node_features, W, attsd, W_edge, attedge, bias2,
      W_out, bout2)
    return (out, gf)
